# baseline (speedup 1.0000x reference)
import functools

import jax
import jax.numpy as jnp
from jax import lax
from jax.experimental import pallas as pl
from jax.experimental.pallas import tpu as pltpu
from jax.experimental.pallas import tpu_sc as plsc

R = 4096
K = 8192
D = 256
NC, NS = 2, 16
NW = NC * NS
ROWS_PER_W = R // NW
BR = 512
NBLK = R // BR
KC = 4
KB = K // KC

_mesh = plsc.VectorSubcoreMesh(core_axis_name="c", subcore_axis_name="s")


def _tc_body(w_ref, idx_ref, rmax, ridx):
    k = pl.program_id(1)
    x = w_ref[...]
    cmax = jnp.max(x, axis=1, keepdims=True)
    ii = lax.broadcasted_iota(jnp.int32, x.shape, 1)
    cand = jnp.where(x == cmax, ii, jnp.int32(KB))
    cidx = jnp.min(cand, axis=1, keepdims=True) + k * KB

    @pl.when(k == 0)
    def _():
        rmax[...] = cmax
        ridx[...] = cidx

    @pl.when(k > 0)
    def _():
        take = cmax > rmax[...]
        rmax[...] = jnp.where(take, cmax, rmax[...])
        ridx[...] = jnp.where(take, cidx, ridx[...])

    @pl.when(k == KC - 1)
    def _():
        idx_ref[0, 0, :] = ridx[...][:, 0]


_tc_argmax = pl.pallas_call(
    _tc_body,
    grid=(NBLK, KC),
    in_specs=[pl.BlockSpec((BR, KB), lambda i, k: (i, k))],
    out_specs=pl.BlockSpec((1, 1, BR), lambda i, k: (i, 0, 0)),
    out_shape=jax.ShapeDtypeStruct((NBLK, 1, BR), jnp.int32),
    scratch_shapes=[
        pltpu.VMEM((BR, 1), jnp.float32),
        pltpu.VMEM((BR, 1), jnp.int32),
    ],
    compiler_params=pltpu.CompilerParams(
        dimension_semantics=("arbitrary", "arbitrary")),
)


@functools.partial(
    pl.kernel,
    out_type=jax.ShapeDtypeStruct((R, D), jnp.float32),
    mesh=_mesh,
    scratch_types=[
        pltpu.VMEM((ROWS_PER_W,), jnp.int32),
        pltpu.VMEM((ROWS_PER_W, D), jnp.float32),
        pltpu.SemaphoreType.DMA,
    ],
    compiler_params=pltpu.CompilerParams(needs_layout_passes=False),
)
def _sc_gather(idx_hbm, cb_hbm, out_hbm, idx_v, rows_v, sem):
    wid = lax.axis_index("s") * NC + lax.axis_index("c")
    base = wid * ROWS_PER_W
    pltpu.sync_copy(idx_hbm.at[pl.ds(base, ROWS_PER_W)], idx_v)
    pltpu.async_copy(cb_hbm.at[idx_v], rows_v, sem).wait()
    pltpu.sync_copy(rows_v, out_hbm.at[pl.ds(base, ROWS_PER_W)])


def kernel(weights, codebook):
    idx = _tc_argmax(weights.reshape(R, K)).reshape(R)
    out = _sc_gather(idx, codebook)
    return out.reshape(weights.shape[0], weights.shape[1], D)


# speedup vs baseline: 1.1835x; 1.1835x over previous
import functools

import jax
import jax.numpy as jnp
from jax import lax
from jax.experimental import pallas as pl
from jax.experimental.pallas import tpu as pltpu
from jax.experimental.pallas import tpu_sc as plsc

R = 4096
K = 8192
D = 256
L = 16
NC, NS = 2, 16
NW = NC * NS
ROWS_PER_W = R // NW
BR = 512
NBLK = R // BR

_mesh = plsc.VectorSubcoreMesh(core_axis_name="c", subcore_axis_name="s")


def _tc_body(w_ref, idx_ref):
    x = w_ref[...]
    m = jnp.max(x, axis=1, keepdims=True)
    ii = lax.broadcasted_iota(jnp.int32, x.shape, 1)
    cand = jnp.where(x == m, ii, jnp.int32(K))
    idx_ref[0, 0, :] = jnp.min(cand, axis=1)


_tc_argmax = pl.pallas_call(
    _tc_body,
    grid=(NBLK,),
    in_specs=[pl.BlockSpec((BR, K), lambda i: (i, 0))],
    out_specs=pl.BlockSpec((1, 1, BR), lambda i: (i, 0, 0)),
    out_shape=jax.ShapeDtypeStruct((NBLK, 1, BR), jnp.int32),
)


@functools.partial(
    pl.kernel,
    out_type=jax.ShapeDtypeStruct((R, D), jnp.float32),
    mesh=_mesh,
    scratch_types=[
        pltpu.VMEM((ROWS_PER_W,), jnp.int32),
        pltpu.VMEM((ROWS_PER_W, D), jnp.float32),
        pltpu.SemaphoreType.DMA,
        pltpu.SemaphoreType.DMA,
        pltpu.SemaphoreType.DMA,
    ],
    compiler_params=pltpu.CompilerParams(needs_layout_passes=False),
)
def _sc_gather(idx_hbm, cb_hbm, out_hbm, idx_v, rows_v, sem, semb, semw):
    wid = lax.axis_index("s") * NC + lax.axis_index("c")
    base = wid * ROWS_PER_W
    half = ROWS_PER_W // 2
    pltpu.sync_copy(idx_hbm.at[pl.ds(base, ROWS_PER_W)], idx_v)
    g0 = pltpu.make_async_copy(
        cb_hbm.at[idx_v.at[pl.ds(0, half)]], rows_v.at[pl.ds(0, half)], sem)
    g1 = pltpu.make_async_copy(
        cb_hbm.at[idx_v.at[pl.ds(half, half)]],
        rows_v.at[pl.ds(half, half)], semb)
    g0.start()
    g1.start()
    g0.wait()
    w0 = pltpu.make_async_copy(
        rows_v.at[pl.ds(0, half)], out_hbm.at[pl.ds(base, half)], semw)
    w0.start()
    g1.wait()
    pltpu.sync_copy(
        rows_v.at[pl.ds(half, half)], out_hbm.at[pl.ds(base + half, half)])
    w0.wait()


def kernel(weights, codebook):
    idx = _tc_argmax(weights.reshape(R, K)).reshape(R)
    out = _sc_gather(idx, codebook)
    return out.reshape(weights.shape[0], weights.shape[1], D)
